# Initial kernel scaffold; baseline (speedup 1.0000x reference)
import jax
import jax.numpy as jnp
from jax.experimental import pallas as pl


def kernel(node_attribute, edge_attribute, edge_length, edge_index, W_proj, b_proj, W_bond, b_bond, b_conv, W_ih, W_hh, b_ih, b_hh, W1, b1, W2, b2, W3, b3):
    raise NotImplementedError("write your pallas kernel here")


# speedup vs baseline: 3.7393x; 3.7393x over previous
import functools

import jax
import jax.numpy as jnp
import numpy as np
from jax import lax
from jax.experimental import pallas as pl
from jax.experimental.pallas import tpu as pltpu
from jax.experimental.pallas import tpu_sc as plsc

N = 10000
E = 160000
DN = 110
DE = 8
DR = 8
H = 16
DH = 1024
NCLS = 206
STEPS = 3
GAMMA = 10.0

P = 8
NP = N // P
EP = E // P

NC = 2
NS = 16
NW = NC * NS
CHUNK = 128
NCHUNK = E // CHUNK
NLOOP = (NCHUNK + NW - 1) // NW
ROWS_PER_TILE = N // NS

def _mm(a, b):
    return jnp.dot(a.astype(jnp.bfloat16), b.astype(jnp.bfloat16),
                   preferred_element_type=jnp.float32)


def _kron8(w):
    return np.kron(np.eye(P, dtype=np.float32), w)


_R8 = _kron8(np.repeat(np.eye(H, dtype=np.float32), H, axis=1)).astype(
    np.float32)
_S8 = _kron8(np.tile(np.eye(H, dtype=np.float32), (H, 1))).astype(
    np.float32)
_SEA = np.zeros((P * DE, 128), np.float32)
_SLEN = np.zeros((P, 128), np.float32)
_MRBF = np.zeros((1, 128), np.float32)
_CRBF = np.zeros((1, 128), np.float32)
for _k in range(P):
    for _a in range(DE):
        _SEA[DE * _k + _a, H * _k + _a] = 1.0
    for _j in range(DR):
        _SLEN[_k, H * _k + DE + _j] = 1.0
        _MRBF[0, H * _k + DE + _j] = 1.0
        _CRBF[0, H * _k + DE + _j] = _j / (DR - 1.0)


def _proj_body(na_ref, wp_ref, bp_ref, h_ref):
    h_ref[...] = jnp.maximum(_mm(na_ref[...], wp_ref[...]) + bp_ref[...], 0.0)


def _proj(na8, Wp8, bp8):
    return pl.pallas_call(
        _proj_body,
        out_shape=jax.ShapeDtypeStruct((NP, 128), jnp.float32),
    )(na8, Wp8, bp8)


_mmx = functools.partial(jnp.dot, precision=jax.lax.Precision.HIGHEST,
                         preferred_element_type=jnp.float32)


def _feat_body(ea_ref, el_ref, sea_ref, slen_ref, m_ref, c_ref, f_ref):
    lenb = _mmx(el_ref[...], slen_ref[...])
    rbf = m_ref[...] * jnp.exp(-GAMMA * (lenb - c_ref[...]) ** 2)
    f_ref[...] = _mmx(ea_ref[...], sea_ref[...]) + rbf


def _feat(eap, lenp):
    BE = 2000
    return pl.pallas_call(
        _feat_body,
        grid=(EP // BE,),
        in_specs=[
            pl.BlockSpec((BE, P * DE), lambda i: (i, 0)),
            pl.BlockSpec((BE, P), lambda i: (i, 0)),
            pl.BlockSpec((P * DE, 128), lambda i: (0, 0)),
            pl.BlockSpec((P, 128), lambda i: (0, 0)),
            pl.BlockSpec((1, 128), lambda i: (0, 0)),
            pl.BlockSpec((1, 128), lambda i: (0, 0)),
        ],
        out_specs=pl.BlockSpec((BE, 128), lambda i: (i, 0)),
        out_shape=jax.ShapeDtypeStruct((EP, 128), jnp.float32),
    )(eap, lenp, _SEA, _SLEN, _MRBF, _CRBF)


def _msg_body(f_ref, xs_ref, r8_ref, wb8_ref, bb8_ref, s8_ref, msg_ref):
    xt = _mm(xs_ref[...], r8_ref[...])
    ew = _mm(f_ref[...], wb8_ref[...]) + bb8_ref[...]
    prod = xt * ew.astype(jnp.bfloat16).astype(jnp.float32)
    msg_ref[...] = _mm(prod, s8_ref[...])


def _msg(featp, xsp, WB8, BB8):
    BE = 1000
    return pl.pallas_call(
        _msg_body,
        grid=(EP // BE,),
        in_specs=[
            pl.BlockSpec((BE, 128), lambda i: (i, 0)),
            pl.BlockSpec((BE, 128), lambda i: (i, 0)),
            pl.BlockSpec((128, P * H * H), lambda i: (0, 0)),
            pl.BlockSpec((128, P * H * H), lambda i: (0, 0)),
            pl.BlockSpec((1, P * H * H), lambda i: (0, 0)),
            pl.BlockSpec((P * H * H, 128), lambda i: (0, 0)),
        ],
        out_specs=pl.BlockSpec((BE, 128), lambda i: (i, 0)),
        out_shape=jax.ShapeDtypeStruct((EP, 128), jnp.float32),
        compiler_params=pltpu.CompilerParams(
            vmem_limit_bytes=64 * 1024 * 1024),
    )(featp, xsp, _R8, WB8, BB8, _S8)


def _gru_body(c0_ref, c1_ref, hid_ref, bc_ref, wir_ref, wiz_ref, win_ref,
              whr_ref, whz_ref, whn_ref, brz_ref, bzz_ref, bin_ref, bhn_ref,
              out_ref):
    x = jnp.maximum(c0_ref[0] + c1_ref[0] + bc_ref[...], 0.0)
    hid = hid_ref[...]
    r = jax.nn.sigmoid(_mm(x, wir_ref[...]) + _mm(hid, whr_ref[...])
                       + brz_ref[...])
    z = jax.nn.sigmoid(_mm(x, wiz_ref[...]) + _mm(hid, whz_ref[...])
                       + bzz_ref[...])
    n = jnp.tanh(_mm(x, win_ref[...]) + bin_ref[...]
                 + r * (_mm(hid, whn_ref[...]) + bhn_ref[...]))
    out_ref[...] = (1.0 - z) * n + z * hid


def _gru(convp, hidden, gw):
    full = lambda i: (0, 0)
    return pl.pallas_call(
        _gru_body,
        grid=(1,),
        in_specs=[
            pl.BlockSpec((1, NP, 128), lambda i: (0, 0, 0)),
            pl.BlockSpec((1, NP, 128), lambda i: (1, 0, 0)),
            pl.BlockSpec((NP, 128), lambda i: (0, 0)),
        ] + [pl.BlockSpec((1, 128), full)]
          + [pl.BlockSpec((128, 128), full)] * 6
          + [pl.BlockSpec((1, 128), full)] * 4,
        out_specs=pl.BlockSpec((NP, 128), lambda i: (0, 0)),
        out_shape=jax.ShapeDtypeStruct((NP, 128), jnp.float32),
    )(convp, convp, hidden, *gw)


def _cls_body(x_ref, h0_ref, w1a_ref, w1b_ref, b1_ref, w2_ref, b2_ref, w3_ref,
              b3_ref, out_ref):
    y = jnp.maximum(_mm(x_ref[...], w1a_ref[...]) +
                    _mm(h0_ref[...], w1b_ref[...]) + b1_ref[...], 0.0)
    y = jnp.maximum(_mm(y, w2_ref[...]) + b2_ref[...], 0.0)
    out_ref[...] = _mm(y, w3_ref[...]) + b3_ref[...]


def _cls(x3, h0, W1, b1, W2, b2, W3, b3):
    BN = 1000
    return pl.pallas_call(
        _cls_body,
        grid=(N // BN,),
        in_specs=[
            pl.BlockSpec((BN, H), lambda i: (i, 0)),
            pl.BlockSpec((BN, H), lambda i: (i, 0)),
            pl.BlockSpec((H, DH), lambda i: (0, 0)),
            pl.BlockSpec((H, DH), lambda i: (0, 0)),
            pl.BlockSpec((1, DH), lambda i: (0, 0)),
            pl.BlockSpec((DH, DH), lambda i: (0, 0)),
            pl.BlockSpec((1, DH), lambda i: (0, 0)),
            pl.BlockSpec((DH, NCLS), lambda i: (0, 0)),
            pl.BlockSpec((1, NCLS), lambda i: (0, 0)),
        ],
        out_specs=pl.BlockSpec((BN, NCLS), lambda i: (i, 0)),
        out_shape=jax.ShapeDtypeStruct((N, NCLS), jnp.float32),
        compiler_params=pltpu.CompilerParams(
            vmem_limit_bytes=64 * 1024 * 1024),
    )(x3, h0, W1[:H], W1[H:], b1.reshape(1, DH), W2, b2.reshape(1, DH), W3,
      b3.reshape(1, NCLS))


_SC_PARAMS = pltpu.CompilerParams(use_tc_tiling_on_sc=False)


def _sc_gather(x_flat, src2):
    mesh = plsc.VectorSubcoreMesh(core_axis_name="c", subcore_axis_name="s")

    @functools.partial(
        pl.kernel, mesh=mesh,
        out_type=jax.ShapeDtypeStruct((E, H), jnp.float32),
        scratch_types=[
            pltpu.VMEM((1, CHUNK), jnp.int32),
            pltpu.VMEM((CHUNK, H), jnp.float32),
        ],
        compiler_params=_SC_PARAMS,
    )
    def k(x_hbm, src_hbm, out_hbm, idx_v, rows_v):
        gw = lax.axis_index("c") * NS + lax.axis_index("s")

        @pl.loop(0, NLOOP)
        def _(j):
            cw = gw + NW * j

            @pl.when(cw < NCHUNK)
            def _():
                pltpu.sync_copy(src_hbm.at[pl.ds(cw, 1)], idx_v)
                pltpu.sync_copy(x_hbm.at[idx_v.at[0]], rows_v)
                pltpu.sync_copy(rows_v, out_hbm.at[pl.ds(cw * CHUNK, CHUNK)])

    return k(x_flat, src2)


def _sc_scatter(msg_flat, dst2):
    mesh = plsc.VectorSubcoreMesh(core_axis_name="c", subcore_axis_name="s")

    @functools.partial(
        pl.kernel, mesh=mesh,
        out_type=jax.ShapeDtypeStruct((NC * N, H), jnp.float32),
        scratch_types=[
            pltpu.VMEM((1, CHUNK), jnp.int32),
            pltpu.VMEM((CHUNK, H), jnp.float32),
            pltpu.VMEM((ROWS_PER_TILE, H), jnp.float32),
            pltpu.VMEM_SHARED((N, H), jnp.float32),
        ],
        compiler_params=_SC_PARAMS,
    )
    def k(msg_hbm, dst_hbm, out_hbm, idx_v, rows_v, zero_v, acc_sh):
        c = lax.axis_index("c")
        s = lax.axis_index("s")
        gw = c * NS + s

        @pl.loop(0, ROWS_PER_TILE)
        def _(i):
            zero_v[i, :] = jnp.zeros((H,), jnp.float32)

        pltpu.sync_copy(zero_v, acc_sh.at[pl.ds(s * ROWS_PER_TILE,
                                                ROWS_PER_TILE)])
        plsc.subcore_barrier()

        @pl.loop(0, NLOOP)
        def _(j):
            cw = gw + NW * j

            @pl.when(cw < NCHUNK)
            def _():
                pltpu.sync_copy(dst_hbm.at[pl.ds(cw, 1)], idx_v)
                pltpu.sync_copy(msg_hbm.at[pl.ds(cw * CHUNK, CHUNK)], rows_v)
                pltpu.sync_copy(rows_v, acc_sh.at[idx_v.at[0]], add=True)

        plsc.subcore_barrier()
        pltpu.sync_copy(
            acc_sh.at[pl.ds(s * ROWS_PER_TILE, ROWS_PER_TILE)],
            out_hbm.at[pl.ds(c * N + s * ROWS_PER_TILE, ROWS_PER_TILE)])

    return k(msg_flat, dst2)


def kernel(node_attribute, edge_attribute, edge_length, edge_index, W_proj,
           b_proj, W_bond, b_bond, b_conv, W_ih, W_hh, b_ih, b_hh, W1, b1, W2,
           b2, W3, b3):
    f32 = jnp.float32
    eye8 = jnp.eye(P, dtype=f32)
    kron8 = lambda w: jnp.kron(eye8, w)
    tile8 = lambda b: jnp.tile(b.reshape(1, -1), (1, P))

    src2 = edge_index[0].reshape(NCHUNK, CHUNK)
    dst2 = edge_index[1].reshape(NCHUNK, CHUNK)
    eap = edge_attribute.reshape(EP, P * DE)
    lenp = edge_length.reshape(EP, P)
    na8 = node_attribute.reshape(NP, P * DN)

    WB8 = kron8(W_bond)
    BB8 = tile8(b_bond)
    Wp8 = kron8(W_proj)
    bp8 = tile8(b_proj)
    gw = ([kron8(W_ih[:, :H]), kron8(W_ih[:, H:2 * H]), kron8(W_ih[:, 2 * H:]),
           kron8(W_hh[:, :H]), kron8(W_hh[:, H:2 * H]), kron8(W_hh[:, 2 * H:])]
          + [tile8(b_ih[:H] + b_hh[:H]),
             tile8(b_ih[H:2 * H] + b_hh[H:2 * H]),
             tile8(b_ih[2 * H:]), tile8(b_hh[2 * H:])])
    bc8 = tile8(b_conv)

    hp = _proj(na8, Wp8, bp8)
    featp = _feat(eap, lenp)

    xp = hp
    hidp = hp
    for _ in range(STEPS):
        xsrc = _sc_gather(xp.reshape(N, H), src2)
        msgp = _msg(featp, xsrc.reshape(EP, 128), WB8, BB8)
        convp = _sc_scatter(msgp.reshape(E, H), dst2)
        hidp = _gru(convp.reshape(2, NP, 128), hidp, [bc8] + gw)
        xp = hidp

    return _cls(xp.reshape(N, H), hp.reshape(N, H), W1, b1, W2, b2, W3, b3)


# speedup vs baseline: 5.1127x; 1.3673x over previous
import functools

import jax
import jax.numpy as jnp
import numpy as np
from jax import lax
from jax.experimental import pallas as pl
from jax.experimental.pallas import tpu as pltpu
from jax.experimental.pallas import tpu_sc as plsc

N = 10000
E = 160000
DN = 110
DE = 8
DR = 8
H = 16
DH = 1024
NCLS = 206
STEPS = 3
GAMMA = 10.0

P = 8
NP = N // P
EP = E // P

NC = 2
NS = 16
NW = NC * NS
CH = 125
NCH_T = 40
SPAN = CH * NCH_T
NCHUNK = E // CH
ROWS_PER_TILE = N // NS

def _mm(a, b):
    return jnp.dot(a.astype(jnp.bfloat16), b.astype(jnp.bfloat16),
                   preferred_element_type=jnp.float32)


def _kron8(w):
    return np.kron(np.eye(P, dtype=np.float32), w)


_R8 = _kron8(np.repeat(np.eye(H, dtype=np.float32), H, axis=1)).astype(
    np.float32)
_S8 = _kron8(np.tile(np.eye(H, dtype=np.float32), (H, 1))).astype(
    np.float32)
_SEA = np.zeros((P * DE, 128), np.float32)
_SLEN = np.zeros((P, 128), np.float32)
_MRBF = np.zeros((1, 128), np.float32)
_CRBF = np.zeros((1, 128), np.float32)
for _k in range(P):
    for _a in range(DE):
        _SEA[DE * _k + _a, H * _k + _a] = 1.0
    for _j in range(DR):
        _SLEN[_k, H * _k + DE + _j] = 1.0
        _MRBF[0, H * _k + DE + _j] = 1.0
        _CRBF[0, H * _k + DE + _j] = _j / (DR - 1.0)


def _proj_body(na_ref, wp_ref, bp_ref, h_ref):
    h_ref[...] = jnp.maximum(_mm(na_ref[...], wp_ref[...]) + bp_ref[...], 0.0)


def _proj(na8, Wp8, bp8):
    return pl.pallas_call(
        _proj_body,
        out_shape=jax.ShapeDtypeStruct((NP, 128), jnp.float32),
    )(na8, Wp8, bp8)


_mmx = functools.partial(jnp.dot, precision=jax.lax.Precision.HIGHEST,
                         preferred_element_type=jnp.float32)


def _feat_body(ea_ref, el_ref, sea_ref, slen_ref, m_ref, c_ref, f_ref):
    lenb = _mmx(el_ref[...], slen_ref[...])
    rbf = m_ref[...] * jnp.exp(-GAMMA * (lenb - c_ref[...]) ** 2)
    f_ref[...] = _mmx(ea_ref[...], sea_ref[...]) + rbf


def _feat(eap, lenp):
    BE = 2000
    return pl.pallas_call(
        _feat_body,
        grid=(EP // BE,),
        in_specs=[
            pl.BlockSpec((BE, P * DE), lambda i: (i, 0)),
            pl.BlockSpec((BE, P), lambda i: (i, 0)),
            pl.BlockSpec((P * DE, 128), lambda i: (0, 0)),
            pl.BlockSpec((P, 128), lambda i: (0, 0)),
            pl.BlockSpec((1, 128), lambda i: (0, 0)),
            pl.BlockSpec((1, 128), lambda i: (0, 0)),
        ],
        out_specs=pl.BlockSpec((BE, 128), lambda i: (i, 0)),
        out_shape=jax.ShapeDtypeStruct((EP, 128), jnp.float32),
    )(eap, lenp, _SEA, _SLEN, _MRBF, _CRBF)


def _msg_body(f_ref, xs_ref, r8_ref, wb8_ref, bb8_ref, s8_ref, msg_ref):
    xt = _mm(xs_ref[...], r8_ref[...])
    ew = _mm(f_ref[...], wb8_ref[...]) + bb8_ref[...]
    prod = xt * ew.astype(jnp.bfloat16).astype(jnp.float32)
    msg_ref[...] = _mm(prod, s8_ref[...])


def _msg(featp, xsp, WB8, BB8):
    BE = 1000
    return pl.pallas_call(
        _msg_body,
        grid=(EP // BE,),
        in_specs=[
            pl.BlockSpec((BE, 128), lambda i: (i, 0)),
            pl.BlockSpec((BE, 128), lambda i: (i, 0)),
            pl.BlockSpec((128, P * H * H), lambda i: (0, 0)),
            pl.BlockSpec((128, P * H * H), lambda i: (0, 0)),
            pl.BlockSpec((1, P * H * H), lambda i: (0, 0)),
            pl.BlockSpec((P * H * H, 128), lambda i: (0, 0)),
        ],
        out_specs=pl.BlockSpec((BE, 128), lambda i: (i, 0)),
        out_shape=jax.ShapeDtypeStruct((EP, 128), jnp.float32),
        compiler_params=pltpu.CompilerParams(
            vmem_limit_bytes=64 * 1024 * 1024),
    )(featp, xsp, _R8, WB8, BB8, _S8)


def _gru_body(c0_ref, c1_ref, hid_ref, bc_ref, wir_ref, wiz_ref, win_ref,
              whr_ref, whz_ref, whn_ref, brz_ref, bzz_ref, bin_ref, bhn_ref,
              out_ref):
    x = jnp.maximum(c0_ref[0] + c1_ref[0] + bc_ref[...], 0.0)
    hid = hid_ref[...]
    r = jax.nn.sigmoid(_mm(x, wir_ref[...]) + _mm(hid, whr_ref[...])
                       + brz_ref[...])
    z = jax.nn.sigmoid(_mm(x, wiz_ref[...]) + _mm(hid, whz_ref[...])
                       + bzz_ref[...])
    n = jnp.tanh(_mm(x, win_ref[...]) + bin_ref[...]
                 + r * (_mm(hid, whn_ref[...]) + bhn_ref[...]))
    out_ref[...] = (1.0 - z) * n + z * hid


def _gru(convp, hidden, gw):
    full = lambda i: (0, 0)
    return pl.pallas_call(
        _gru_body,
        grid=(1,),
        in_specs=[
            pl.BlockSpec((1, NP, 128), lambda i: (0, 0, 0)),
            pl.BlockSpec((1, NP, 128), lambda i: (1, 0, 0)),
            pl.BlockSpec((NP, 128), lambda i: (0, 0)),
        ] + [pl.BlockSpec((1, 128), full)]
          + [pl.BlockSpec((128, 128), full)] * 6
          + [pl.BlockSpec((1, 128), full)] * 4,
        out_specs=pl.BlockSpec((NP, 128), lambda i: (0, 0)),
        out_shape=jax.ShapeDtypeStruct((NP, 128), jnp.float32),
    )(convp, convp, hidden, *gw)


def _cls_body(x_ref, h0_ref, w1a_ref, w1b_ref, b1_ref, w2_ref, b2_ref, w3_ref,
              b3_ref, out_ref):
    y = jnp.maximum(_mm(x_ref[...], w1a_ref[...]) +
                    _mm(h0_ref[...], w1b_ref[...]) + b1_ref[...], 0.0)
    y = jnp.maximum(_mm(y, w2_ref[...]) + b2_ref[...], 0.0)
    out_ref[...] = _mm(y, w3_ref[...]) + b3_ref[...]


def _cls(x3, h0, W1, b1, W2, b2, W3, b3):
    BN = 1000
    return pl.pallas_call(
        _cls_body,
        grid=(N // BN,),
        in_specs=[
            pl.BlockSpec((BN, H), lambda i: (i, 0)),
            pl.BlockSpec((BN, H), lambda i: (i, 0)),
            pl.BlockSpec((H, DH), lambda i: (0, 0)),
            pl.BlockSpec((H, DH), lambda i: (0, 0)),
            pl.BlockSpec((1, DH), lambda i: (0, 0)),
            pl.BlockSpec((DH, DH), lambda i: (0, 0)),
            pl.BlockSpec((1, DH), lambda i: (0, 0)),
            pl.BlockSpec((DH, NCLS), lambda i: (0, 0)),
            pl.BlockSpec((1, NCLS), lambda i: (0, 0)),
        ],
        out_specs=pl.BlockSpec((BN, NCLS), lambda i: (i, 0)),
        out_shape=jax.ShapeDtypeStruct((N, NCLS), jnp.float32),
        compiler_params=pltpu.CompilerParams(
            vmem_limit_bytes=64 * 1024 * 1024),
    )(x3, h0, W1[:H], W1[H:], b1.reshape(1, DH), W2, b2.reshape(1, DH), W3,
      b3.reshape(1, NCLS))


_SC_PARAMS = pltpu.CompilerParams(use_tc_tiling_on_sc=False)


def _sc_gather(x_flat, src3):
    mesh = plsc.VectorSubcoreMesh(core_axis_name="c", subcore_axis_name="s")

    @functools.partial(
        pl.kernel, mesh=mesh,
        out_type=jax.ShapeDtypeStruct((E, H), jnp.float32),
        scratch_types=[
            pltpu.VMEM((NCH_T, CH), jnp.int32),
            pltpu.VMEM((SPAN, H), jnp.float32),
            pltpu.SemaphoreType.DMA,
        ],
        compiler_params=_SC_PARAMS,
    )
    def k(x_hbm, src_hbm, out_hbm, idx_v, rows_v, sem):
        t = lax.axis_index("c") * NS + lax.axis_index("s")
        pltpu.sync_copy(src_hbm.at[pl.ds(t * NCH_T, NCH_T)], idx_v)

        @pl.loop(0, NCH_T)
        def _(j):
            pltpu.async_copy(x_hbm.at[idx_v.at[j]],
                             rows_v.at[pl.ds(j * CH, CH)], sem)

        pltpu.make_async_copy(x_hbm.at[pl.ds(0, SPAN)], rows_v, sem).wait()
        pltpu.sync_copy(rows_v, out_hbm.at[pl.ds(t * SPAN, SPAN)])

    return k(x_flat, src3)


def _sc_scatter(msg_flat, dst3):
    mesh = plsc.VectorSubcoreMesh(core_axis_name="c", subcore_axis_name="s")

    @functools.partial(
        pl.kernel, mesh=mesh,
        out_type=jax.ShapeDtypeStruct((NC * N, H), jnp.float32),
        scratch_types=[
            pltpu.VMEM((NCH_T, CH), jnp.int32),
            pltpu.VMEM((SPAN, H), jnp.float32),
            pltpu.VMEM((ROWS_PER_TILE, H), jnp.float32),
            pltpu.VMEM_SHARED((N, H), jnp.float32),
            pltpu.SemaphoreType.DMA,
        ],
        compiler_params=_SC_PARAMS,
    )
    def k(msg_hbm, dst_hbm, out_hbm, idx_v, rows_v, zero_v, acc_sh, sem):
        c = lax.axis_index("c")
        s = lax.axis_index("s")
        t = c * NS + s

        pltpu.async_copy(msg_hbm.at[pl.ds(t * SPAN, SPAN)], rows_v, sem)
        pltpu.sync_copy(dst_hbm.at[pl.ds(t * NCH_T, NCH_T)], idx_v)

        @pl.loop(0, ROWS_PER_TILE)
        def _(i):
            zero_v[i, :] = jnp.zeros((H,), jnp.float32)

        pltpu.sync_copy(zero_v, acc_sh.at[pl.ds(s * ROWS_PER_TILE,
                                                ROWS_PER_TILE)])
        plsc.subcore_barrier()
        pltpu.make_async_copy(msg_hbm.at[pl.ds(0, SPAN)], rows_v, sem).wait()

        @pl.loop(0, NCH_T)
        def _(j):
            pltpu.async_copy(rows_v.at[pl.ds(j * CH, CH)],
                             acc_sh.at[idx_v.at[j]], sem, add=True)

        pltpu.make_async_copy(msg_hbm.at[pl.ds(0, SPAN)], rows_v, sem).wait()
        plsc.subcore_barrier()
        pltpu.sync_copy(
            acc_sh.at[pl.ds(s * ROWS_PER_TILE, ROWS_PER_TILE)],
            out_hbm.at[pl.ds(c * N + s * ROWS_PER_TILE, ROWS_PER_TILE)])

    return k(msg_flat, dst3)


def kernel(node_attribute, edge_attribute, edge_length, edge_index, W_proj,
           b_proj, W_bond, b_bond, b_conv, W_ih, W_hh, b_ih, b_hh, W1, b1, W2,
           b2, W3, b3):
    f32 = jnp.float32
    eye8 = jnp.eye(P, dtype=f32)
    kron8 = lambda w: jnp.kron(eye8, w)
    tile8 = lambda b: jnp.tile(b.reshape(1, -1), (1, P))

    src3 = edge_index[0].reshape(NCHUNK, CH)
    dst3 = edge_index[1].reshape(NCHUNK, CH)
    eap = edge_attribute.reshape(EP, P * DE)
    lenp = edge_length.reshape(EP, P)
    na8 = node_attribute.reshape(NP, P * DN)

    WB8 = kron8(W_bond)
    BB8 = tile8(b_bond)
    Wp8 = kron8(W_proj)
    bp8 = tile8(b_proj)
    gw = ([kron8(W_ih[:, :H]), kron8(W_ih[:, H:2 * H]), kron8(W_ih[:, 2 * H:]),
           kron8(W_hh[:, :H]), kron8(W_hh[:, H:2 * H]), kron8(W_hh[:, 2 * H:])]
          + [tile8(b_ih[:H] + b_hh[:H]),
             tile8(b_ih[H:2 * H] + b_hh[H:2 * H]),
             tile8(b_ih[2 * H:]), tile8(b_hh[2 * H:])])
    bc8 = tile8(b_conv)

    hp = _proj(na8, Wp8, bp8)
    featp = _feat(eap, lenp)

    xp = hp
    hidp = hp
    for _ in range(STEPS):
        xsrc = _sc_gather(xp.reshape(N, H), src3)
        msgp = _msg(featp, xsrc.reshape(EP, 128), WB8, BB8)
        convp = _sc_scatter(msgp.reshape(E, H), dst3)
        hidp = _gru(convp.reshape(2, NP, 128), hidp, [bc8] + gw)
        xp = hidp

    return _cls(xp.reshape(N, H), hp.reshape(N, H), W1, b1, W2, b2, W3, b3)
